# baseline (speedup 1.0000x reference)
import jax
import jax.numpy as jnp
from jax import lax
from jax.experimental import pallas as pl
from jax.experimental.pallas import tpu as pltpu
from jax.experimental.pallas import tpu_sc as plsc

N = 10000
E = 320000
D = 128
K = 10
ALPHA = 0.1

NPAD = 10240
EPAD = 327680
EG = EPAD // 64
NSUB = 16
EGT = EG // NSUB
NPT = NPAD // NSUB
BC = 8
NCH = NPT // BC
BG = 16
NBUF = 4


def _rsqrt16(x):
    i = lax.bitcast_convert_type(x, jnp.int32)
    y = lax.bitcast_convert_type(
        jnp.full((16,), 0x5F3759DF, jnp.int32) - (i >> 1), jnp.float32)
    for _ in range(3):
        y = y * (1.5 - 0.5 * x * y * y)
    return y


def _sc_body(inp_hbm, srcp_hbm, dstp_hbm,
             hA_hbm, hB_hbm, g_hbm,
             m_sh, deg_sh,
             src16, dst16, src16b, dst16b,
             st0, st1, st2, st3,
             mb, h0b, hb, gb, zb, zf, norm_v, ones_v,
             gs0, gs1, gs2, gs3, as0, as1, as2, as3,
             is0, is1, is2, is3):
    sid = lax.axis_index("s")
    gbase = sid * EGT
    nbase = sid * NPT
    st = [st0, st1, st2, st3]
    gsem = [gs0, gs1, gs2, gs3]
    asem = [as0, as1, as2, as3]

    def _fill_z(i, c):
        r = i // 8
        col = i % 8
        zb[r, pl.ds(col * 16, 16)] = jnp.zeros((16,), jnp.float32)
        return c
    lax.fori_loop(0, BC * 8, _fill_z, None)

    def _fill_zf(i, c):
        zf[pl.ds(i * 16, 16)] = jnp.zeros((16,), jnp.float32)
        return c
    lax.fori_loop(0, NPT // 16, _fill_zf, None)

    def _fill_1(i, c):
        ones_v[pl.ds(i * 16, 16)] = jnp.ones((16,), jnp.float32)
        return c
    lax.fori_loop(0, 4, _fill_1, None)

    pltpu.sync_copy(zf, deg_sh.at[pl.ds(nbase, NPT)])
    plsc.subcore_barrier()

    def _deg_t(t, c):
        pltpu.sync_copy(dstp_hbm.at[pl.ds(gbase + t * BG, BG)], dst16)

        def _inner(q, c2):
            pltpu.sync_copy(ones_v, deg_sh.at[dst16.at[q]], add=True)
            return c2
        lax.fori_loop(0, BG, _inner, None)
        return c
    lax.fori_loop(0, EGT // BG, _deg_t, None)
    plsc.subcore_barrier()

    pltpu.sync_copy(deg_sh.at[pl.ds(nbase, NPT)], norm_v)

    def _norm(c, carry):
        d = jnp.maximum(norm_v[pl.ds(c * 16, 16)], 1.0)
        norm_v[pl.ds(c * 16, 16)] = _rsqrt16(d)
        return carry
    lax.fori_loop(0, NPT // 16, _norm, None)

    def _init(c, carry):
        row0 = c * BC
        pltpu.sync_copy(inp_hbm.at[pl.ds(nbase + row0, BC)], h0b)
        pltpu.sync_copy(zb, m_sh.at[pl.ds(nbase + row0, BC)])

        def _rows(r, cr):
            nb = plsc.load_gather(norm_v,
                                  [jnp.full((16,), row0 + r, jnp.int32)])
            for cc in range(8):
                s = pl.ds(cc * 16, 16)
                gb[r, s] = nb * h0b[r, s]
            return cr
        lax.fori_loop(0, BC, _rows, None)
        pltpu.sync_copy(gb, g_hbm.at[pl.ds(nbase + row0, BC)])
        return carry
    lax.fori_loop(0, NCH, _init, None)
    plsc.subcore_barrier()

    def _step(h0_hbm, hout_hbm, write_h, write_g):
        def _block(srcb, dstb):
            gc = [None] * BG
            ac = [None] * BG
            for q in range(NBUF):
                gc[q] = pltpu.async_copy(
                    g_hbm.at[srcb.at[q]], st[q], gsem[q])
            for q in range(BG):
                b = q % NBUF
                gc[q].wait()
                ac[q] = pltpu.async_copy(
                    st[b], m_sh.at[dstb.at[q]], asem[b], add=True)
                nxt = q + 2
                if q >= 2 and nxt < BG:
                    bb = nxt % NBUF
                    ac[q - 2].wait()
                    gc[nxt] = pltpu.async_copy(
                        g_hbm.at[srcb.at[nxt]], st[bb], gsem[bb])
            for q in range(BG - NBUF, BG):
                ac[q].wait()

        def _pf_idx(blk, sbuf, dbuf, ssem, dsem):
            base = gbase + jnp.minimum(blk, EGT // BG - 1) * BG
            pltpu.async_copy(srcp_hbm.at[pl.ds(base, BG)], sbuf, ssem)
            pltpu.async_copy(dstp_hbm.at[pl.ds(base, BG)], dbuf, dsem)

        def _wait_idx(sbuf, dbuf, ssem, dsem):
            pltpu.make_async_copy(srcp_hbm.at[pl.ds(gbase, BG)],
                                  sbuf, ssem).wait()
            pltpu.make_async_copy(dstp_hbm.at[pl.ds(gbase, BG)],
                                  dbuf, dsem).wait()

        _pf_idx(0, src16, dst16, is0, is1)

        def _edges2(t2, carry):
            _wait_idx(src16, dst16, is0, is1)
            _pf_idx(2 * t2 + 1, src16b, dst16b, is2, is3)
            _block(src16, dst16)
            _wait_idx(src16b, dst16b, is2, is3)
            _pf_idx(2 * t2 + 2, src16, dst16, is0, is1)
            _block(src16b, dst16b)
            return carry
        lax.fori_loop(0, EGT // BG // 2, _edges2, None)
        _wait_idx(src16, dst16, is0, is1)
        plsc.subcore_barrier()

        def _blend(c, carry):
            row0 = c * BC
            pltpu.sync_copy(m_sh.at[pl.ds(nbase + row0, BC)], mb)
            pltpu.sync_copy(zb, m_sh.at[pl.ds(nbase + row0, BC)])
            pltpu.sync_copy(h0_hbm.at[pl.ds(nbase + row0, BC)], h0b)

            def _rows(r, cr):
                nb = plsc.load_gather(norm_v,
                                      [jnp.full((16,), row0 + r, jnp.int32)])
                for cc in range(8):
                    s = pl.ds(cc * 16, 16)
                    h16 = 0.9 * nb * mb[r, s] + 0.1 * h0b[r, s]
                    hb[r, s] = h16
                    gb[r, s] = nb * h16
                return cr
            lax.fori_loop(0, BC, _rows, None)

            @pl.when(write_h)
            def _():
                pltpu.sync_copy(hb, hout_hbm.at[pl.ds(nbase + row0, BC)])

            @pl.when(write_g)
            def _():
                pltpu.sync_copy(gb, g_hbm.at[pl.ds(nbase + row0, BC)])
            return carry
        lax.fori_loop(0, NCH, _blend, None)
        plsc.subcore_barrier()

    def _pass1(k, carry):
        _step(inp_hbm, hA_hbm, k == K - 1, True)
        return carry
    lax.fori_loop(0, K, _pass1, None)

    def _pass2(k, carry):
        _step(hA_hbm, hB_hbm, k == K - 1, k < K - 1)
        return carry
    lax.fori_loop(0, K, _pass2, None)


def _appnp_sc(inp, srcp, dstp):
    mesh = plsc.VectorSubcoreMesh(
        core_axis_name="c", subcore_axis_name="s", num_cores=1)
    f32 = jnp.float32
    out_type = (
        jax.ShapeDtypeStruct((NPAD, D), f32),
        jax.ShapeDtypeStruct((NPAD, D), f32),
        jax.ShapeDtypeStruct((NPAD, D), f32),
    )
    scratch = [
        pltpu.VMEM_SHARED((NPAD, D), f32),
        pltpu.VMEM_SHARED((NPAD,), f32),
        pltpu.VMEM((BG, 64), jnp.int32),
        pltpu.VMEM((BG, 64), jnp.int32),
        pltpu.VMEM((BG, 64), jnp.int32),
        pltpu.VMEM((BG, 64), jnp.int32),
        pltpu.VMEM((64, D), f32),
        pltpu.VMEM((64, D), f32),
        pltpu.VMEM((64, D), f32),
        pltpu.VMEM((64, D), f32),
        pltpu.VMEM((BC, D), f32),
        pltpu.VMEM((BC, D), f32),
        pltpu.VMEM((BC, D), f32),
        pltpu.VMEM((BC, D), f32),
        pltpu.VMEM((BC, D), f32),
        pltpu.VMEM((NPT,), f32),
        pltpu.VMEM((NPT,), f32),
        pltpu.VMEM((64,), f32),
        pltpu.SemaphoreType.DMA,
        pltpu.SemaphoreType.DMA,
        pltpu.SemaphoreType.DMA,
        pltpu.SemaphoreType.DMA,
        pltpu.SemaphoreType.DMA,
        pltpu.SemaphoreType.DMA,
        pltpu.SemaphoreType.DMA,
        pltpu.SemaphoreType.DMA,
        pltpu.SemaphoreType.DMA,
        pltpu.SemaphoreType.DMA,
        pltpu.SemaphoreType.DMA,
        pltpu.SemaphoreType.DMA,
    ]
    run = pl.kernel(_sc_body, out_type=out_type, mesh=mesh,
                    scratch_types=scratch,
                    compiler_params=pltpu.CompilerParams(
                        needs_layout_passes=False))
    _, hB, _ = run(inp, srcp, dstp)
    return hB


def _mlp_body(x_ref, w1, b1, w2, b2, wh1, bh1, wh2, bh2, hl_ref, o_ref):
    x = x_ref[...]
    h1 = jnp.maximum(x @ w1[...] + b1[...], 0.0)
    hl = h1 @ w2[...] + b2[...]
    hl_ref[...] = hl
    t = jnp.maximum(hl @ wh1[...] + bh1[...], 0.0)
    o_ref[...] = t @ wh2[...] + bh2[...]


def _mlp_tc(h, W1, b1, W2, b2, Wh1, bh1, Wh2, bh2):
    BLK = 1000
    grid = (N // BLK,)

    def full(shape):
        return pl.BlockSpec(shape, lambda i: (0, 0))

    hl, out0 = pl.pallas_call(
        _mlp_body,
        grid=grid,
        in_specs=[
            pl.BlockSpec((BLK, 128), lambda i: (i, 0)),
            full((128, 128)), full((1, 128)),
            full((128, 128)), full((1, 128)),
            full((128, 64)), full((1, 64)),
            full((64, 40)), full((1, 40)),
        ],
        out_specs=[
            pl.BlockSpec((BLK, 128), lambda i: (i, 0)),
            pl.BlockSpec((BLK, 40), lambda i: (i, 0)),
        ],
        out_shape=[
            jax.ShapeDtypeStruct((N, 128), jnp.float32),
            jax.ShapeDtypeStruct((N, 40), jnp.float32),
        ],
    )(h, W1, b1.reshape(1, -1), W2, b2.reshape(1, -1),
      Wh1, bh1.reshape(1, -1), Wh2, bh2.reshape(1, -1))
    return hl, out0


@jax.jit
def kernel(input_feat, edge_index, W1, b1, W2, b2, Wh1, bh1, Wh2, bh2):
    inp = jnp.pad(input_feat, ((0, NPAD - N), (0, 0)))
    srcp = jnp.pad(edge_index[0], (0, EPAD - E),
                   constant_values=N).reshape(EG, 64)
    dstp = jnp.pad(edge_index[1], (0, EPAD - E),
                   constant_values=N).reshape(EG, 64)
    h = _appnp_sc(inp, srcp, dstp)[:N]
    h_last, out0 = _mlp_tc(h, W1, b1, W2, b2, Wh1, bh1, Wh2, bh2)
    return (out0, h_last)


# speedup vs baseline: 1.0119x; 1.0119x over previous
import jax
import jax.numpy as jnp
from jax import lax
from jax.experimental import pallas as pl
from jax.experimental.pallas import tpu as pltpu
from jax.experimental.pallas import tpu_sc as plsc

N = 10000
E = 320000
D = 128
K = 10
ALPHA = 0.1

NPAD = 10240
EPAD = 327680
EG = EPAD // 64
NSUB = 16
EGT = EG // NSUB
NPT = NPAD // NSUB
BC = 16
NCH = NPT // BC
BG = 16
NBUF = 4


def _rsqrt16(x):
    i = lax.bitcast_convert_type(x, jnp.int32)
    y = lax.bitcast_convert_type(
        jnp.full((16,), 0x5F3759DF, jnp.int32) - (i >> 1), jnp.float32)
    for _ in range(3):
        y = y * (1.5 - 0.5 * x * y * y)
    return y


def _sc_body(inp_hbm, srcp_hbm, dstp_hbm,
             hA_hbm, hB_hbm, g_hbm,
             m_sh, deg_sh,
             src16, dst16, st0, st1, st2, st3,
             mb, h0b, hb, gb, zb, zf, norm_v, ones_v,
             gs0, gs1, gs2, gs3, as0, as1, as2, as3):
    sid = lax.axis_index("s")
    gbase = sid * EGT
    nbase = sid * NPT
    st = [st0, st1, st2, st3]
    gsem = [gs0, gs1, gs2, gs3]
    asem = [as0, as1, as2, as3]

    def _fill_z(i, c):
        r = i // 8
        col = i % 8
        zb[r, pl.ds(col * 16, 16)] = jnp.zeros((16,), jnp.float32)
        return c
    lax.fori_loop(0, BC * 8, _fill_z, None)

    def _fill_zf(i, c):
        zf[pl.ds(i * 16, 16)] = jnp.zeros((16,), jnp.float32)
        return c
    lax.fori_loop(0, NPT // 16, _fill_zf, None)

    def _fill_1(i, c):
        ones_v[pl.ds(i * 16, 16)] = jnp.ones((16,), jnp.float32)
        return c
    lax.fori_loop(0, 4, _fill_1, None)

    pltpu.sync_copy(zf, deg_sh.at[pl.ds(nbase, NPT)])
    plsc.subcore_barrier()

    def _deg_t(t, c):
        pltpu.sync_copy(dstp_hbm.at[pl.ds(gbase + t * BG, BG)], dst16)

        def _inner(q, c2):
            pltpu.sync_copy(ones_v, deg_sh.at[dst16.at[q]], add=True)
            return c2
        lax.fori_loop(0, BG, _inner, None)
        return c
    lax.fori_loop(0, EGT // BG, _deg_t, None)
    plsc.subcore_barrier()

    pltpu.sync_copy(deg_sh.at[pl.ds(nbase, NPT)], norm_v)

    def _norm(c, carry):
        d = jnp.maximum(norm_v[pl.ds(c * 16, 16)], 1.0)
        norm_v[pl.ds(c * 16, 16)] = _rsqrt16(d)
        return carry
    lax.fori_loop(0, NPT // 16, _norm, None)

    def _init(c, carry):
        row0 = c * BC
        pltpu.sync_copy(inp_hbm.at[pl.ds(nbase + row0, BC)], h0b)
        pltpu.sync_copy(zb, m_sh.at[pl.ds(nbase + row0, BC)])

        def _rows(r, cr):
            nb = plsc.load_gather(norm_v,
                                  [jnp.full((16,), row0 + r, jnp.int32)])
            for cc in range(8):
                s = pl.ds(cc * 16, 16)
                gb[r, s] = nb * h0b[r, s]
            return cr
        lax.fori_loop(0, BC, _rows, None)
        pltpu.sync_copy(gb, g_hbm.at[pl.ds(nbase + row0, BC)])
        return carry
    lax.fori_loop(0, NCH, _init, None)
    plsc.subcore_barrier()

    def _step(h0_hbm, hout_hbm, write_h, write_g):
        def _edges(t, carry):
            base = gbase + t * BG
            pltpu.sync_copy(srcp_hbm.at[pl.ds(base, BG)], src16)
            pltpu.sync_copy(dstp_hbm.at[pl.ds(base, BG)], dst16)

            gc = [None] * BG
            ac = [None] * BG
            for q in range(NBUF):
                gc[q] = pltpu.async_copy(
                    g_hbm.at[src16.at[q]], st[q], gsem[q])
            for q in range(BG):
                b = q % NBUF
                gc[q].wait()
                ac[q] = pltpu.async_copy(
                    st[b], m_sh.at[dst16.at[q]], asem[b], add=True)
                nxt = q + 2
                if q >= 2 and nxt < BG:
                    bb = nxt % NBUF
                    ac[q - 2].wait()
                    gc[nxt] = pltpu.async_copy(
                        g_hbm.at[src16.at[nxt]], st[bb], gsem[bb])
            for q in range(BG - NBUF, BG):
                ac[q].wait()
            return carry
        lax.fori_loop(0, EGT // BG, _edges, None)
        plsc.subcore_barrier()

        def _blend(c, carry):
            row0 = c * BC
            pltpu.sync_copy(m_sh.at[pl.ds(nbase + row0, BC)], mb)
            pltpu.sync_copy(zb, m_sh.at[pl.ds(nbase + row0, BC)])
            pltpu.sync_copy(h0_hbm.at[pl.ds(nbase + row0, BC)], h0b)

            def _rows(r, cr):
                nb = plsc.load_gather(norm_v,
                                      [jnp.full((16,), row0 + r, jnp.int32)])
                for cc in range(8):
                    s = pl.ds(cc * 16, 16)
                    h16 = 0.9 * nb * mb[r, s] + 0.1 * h0b[r, s]
                    hb[r, s] = h16
                    gb[r, s] = nb * h16
                return cr
            lax.fori_loop(0, BC, _rows, None)

            @pl.when(write_h)
            def _():
                pltpu.sync_copy(hb, hout_hbm.at[pl.ds(nbase + row0, BC)])

            @pl.when(write_g)
            def _():
                pltpu.sync_copy(gb, g_hbm.at[pl.ds(nbase + row0, BC)])
            return carry
        lax.fori_loop(0, NCH, _blend, None)
        plsc.subcore_barrier()

    def _pass1(k, carry):
        _step(inp_hbm, hA_hbm, k == K - 1, True)
        return carry
    lax.fori_loop(0, K, _pass1, None)

    def _pass2(k, carry):
        _step(hA_hbm, hB_hbm, k == K - 1, k < K - 1)
        return carry
    lax.fori_loop(0, K, _pass2, None)


def _appnp_sc(inp, srcp, dstp):
    mesh = plsc.VectorSubcoreMesh(
        core_axis_name="c", subcore_axis_name="s", num_cores=1)
    f32 = jnp.float32
    out_type = (
        jax.ShapeDtypeStruct((NPAD, D), f32),
        jax.ShapeDtypeStruct((NPAD, D), f32),
        jax.ShapeDtypeStruct((NPAD, D), f32),
    )
    scratch = [
        pltpu.VMEM_SHARED((NPAD, D), f32),
        pltpu.VMEM_SHARED((NPAD,), f32),
        pltpu.VMEM((BG, 64), jnp.int32),
        pltpu.VMEM((BG, 64), jnp.int32),
        pltpu.VMEM((64, D), f32),
        pltpu.VMEM((64, D), f32),
        pltpu.VMEM((64, D), f32),
        pltpu.VMEM((64, D), f32),
        pltpu.VMEM((BC, D), f32),
        pltpu.VMEM((BC, D), f32),
        pltpu.VMEM((BC, D), f32),
        pltpu.VMEM((BC, D), f32),
        pltpu.VMEM((BC, D), f32),
        pltpu.VMEM((NPT,), f32),
        pltpu.VMEM((NPT,), f32),
        pltpu.VMEM((64,), f32),
        pltpu.SemaphoreType.DMA,
        pltpu.SemaphoreType.DMA,
        pltpu.SemaphoreType.DMA,
        pltpu.SemaphoreType.DMA,
        pltpu.SemaphoreType.DMA,
        pltpu.SemaphoreType.DMA,
        pltpu.SemaphoreType.DMA,
        pltpu.SemaphoreType.DMA,
    ]
    run = pl.kernel(_sc_body, out_type=out_type, mesh=mesh,
                    scratch_types=scratch,
                    compiler_params=pltpu.CompilerParams(
                        needs_layout_passes=False))
    _, hB, _ = run(inp, srcp, dstp)
    return hB


def _mlp_body(x_ref, w1, b1, w2, b2, wh1, bh1, wh2, bh2, hl_ref, o_ref):
    x = x_ref[...]
    h1 = jnp.maximum(x @ w1[...] + b1[...], 0.0)
    hl = h1 @ w2[...] + b2[...]
    hl_ref[...] = hl
    t = jnp.maximum(hl @ wh1[...] + bh1[...], 0.0)
    o_ref[...] = t @ wh2[...] + bh2[...]


def _mlp_tc(h, W1, b1, W2, b2, Wh1, bh1, Wh2, bh2):
    BLK = 1000
    grid = (N // BLK,)

    def full(shape):
        return pl.BlockSpec(shape, lambda i: (0, 0))

    hl, out0 = pl.pallas_call(
        _mlp_body,
        grid=grid,
        in_specs=[
            pl.BlockSpec((BLK, 128), lambda i: (i, 0)),
            full((128, 128)), full((1, 128)),
            full((128, 128)), full((1, 128)),
            full((128, 64)), full((1, 64)),
            full((64, 40)), full((1, 40)),
        ],
        out_specs=[
            pl.BlockSpec((BLK, 128), lambda i: (i, 0)),
            pl.BlockSpec((BLK, 40), lambda i: (i, 0)),
        ],
        out_shape=[
            jax.ShapeDtypeStruct((N, 128), jnp.float32),
            jax.ShapeDtypeStruct((N, 40), jnp.float32),
        ],
    )(h, W1, b1.reshape(1, -1), W2, b2.reshape(1, -1),
      Wh1, bh1.reshape(1, -1), Wh2, bh2.reshape(1, -1))
    return hl, out0


@jax.jit
def kernel(input_feat, edge_index, W1, b1, W2, b2, Wh1, bh1, Wh2, bh2):
    inp = jnp.pad(input_feat, ((0, NPAD - N), (0, 0)))
    srcp = jnp.pad(edge_index[0], (0, EPAD - E),
                   constant_values=N).reshape(EG, 64)
    dstp = jnp.pad(edge_index[1], (0, EPAD - E),
                   constant_values=N).reshape(EG, 64)
    h = _appnp_sc(inp, srcp, dstp)[:N]
    h_last, out0 = _mlp_tc(h, W1, b1, W2, b2, Wh1, bh1, Wh2, bh2)
    return (out0, h_last)
